# baseline (speedup 1.0000x reference)
import jax
import jax.numpy as jnp
from jax.experimental import pallas as pl

_NPOINT = 1024
_K = 32


def _fps(xyz, npoint):
    B, N, _ = xyz.shape
    batch_idx = jnp.arange(B)

    def body(i, state):
        centroids, distance, farthest = state
        centroids = centroids.at[:, i].set(farthest)
        centroid = xyz[batch_idx, farthest][:, None, :]
        dist = jnp.sum((xyz - centroid) ** 2, axis=-1)
        distance = jnp.minimum(distance, dist)
        farthest = jnp.argmax(distance, axis=-1).astype(jnp.int32)
        return (centroids, distance, farthest)

    centroids0 = jnp.zeros((B, npoint), dtype=jnp.int32)
    distance0 = jnp.full((B, N), 1e10, dtype=xyz.dtype)
    farthest0 = jnp.zeros((B,), dtype=jnp.int32)
    centroids, _, _ = jax.lax.fori_loop(0, npoint, body, (centroids0, distance0, farthest0))
    return centroids


def _copy_kernel(x_ref, o_ref):
    o_ref[...] = x_ref[...]


def kernel(xyz, features, W0, g0, b0, W1, g1, b1, W2, g2, b2):
    B, N, _ = xyz.shape
    fps_idx = _fps(xyz, _NPOINT)
    new_xyz = jax.vmap(lambda p, i: p[i])(xyz, fps_idx)

    s2 = jnp.sum(new_xyz ** 2, axis=-1)[:, :, None]
    d2 = jnp.sum(xyz ** 2, axis=-1)[:, None, :]
    cross = jnp.einsum('bnd,bmd->bnm', new_xyz, xyz)
    dist = jnp.maximum(s2 + d2 - 2.0 * cross, 0.0)
    _, group_idx = jax.lax.top_k(-dist, _K)

    grouped_xyz = jax.vmap(lambda p, i: p[i])(xyz, group_idx)
    grouped_xyz_norm = grouped_xyz - new_xyz[:, :, None, :]
    feats_t = jnp.transpose(features, (0, 2, 1))
    grouped_features = jax.vmap(lambda p, i: p[i])(feats_t, group_idx)
    grouped_features = jnp.transpose(grouped_features, (0, 3, 1, 2))
    grouped_input = jnp.concatenate(
        [jnp.transpose(grouped_xyz_norm, (0, 3, 1, 2)), grouped_features], axis=1
    )

    def mlp_layer(x, W, g, b):
        y = jnp.einsum('bcsk,oc->bosk', x, W)
        mean = jnp.mean(y, axis=(0, 2, 3), keepdims=True)
        var = jnp.var(y, axis=(0, 2, 3), keepdims=True)
        y = (y - mean) / jnp.sqrt(var + 1e-5)
        y = y * g[None, :, None, None] + b[None, :, None, None]
        return jax.nn.relu(y)

    y = mlp_layer(grouped_input, W0, g0, b0)
    y = mlp_layer(y, W1, g1, b1)
    y = mlp_layer(y, W2, g2, b2)
    new_features = jnp.max(y, axis=-1)

    new_features = pl.pallas_call(
        _copy_kernel,
        out_shape=jax.ShapeDtypeStruct(new_features.shape, new_features.dtype),
    )(new_features)
    return (new_xyz, new_features)


# speedup vs baseline: 1.7474x; 1.7474x over previous
import jax
import jax.numpy as jnp
from jax.experimental import pallas as pl
from jax.experimental.pallas import tpu as pltpu

_B = 4
_N = 8192
_NPOINT = 1024
_K = 32


def _fps_kernel(x_ref, y_ref, z_ref, cx_ref, cy_ref, cz_ref, dist_ref):
    X = x_ref[...]
    Y = y_ref[...]
    Z = z_ref[...]
    dist_ref[...] = jnp.full((_B, _N), 1e10, dtype=jnp.float32)
    iota = jax.lax.broadcasted_iota(jnp.int32, (_B, _N), 1)

    def body(i, carry):
        cx, cy, cz = carry
        cx_ref[pl.ds(i, 1), :] = cx.reshape(1, _B)
        cy_ref[pl.ds(i, 1), :] = cy.reshape(1, _B)
        cz_ref[pl.ds(i, 1), :] = cz.reshape(1, _B)
        dx = X - cx
        dy = Y - cy
        dz = Z - cz
        d = dx * dx + dy * dy + dz * dz
        dmin = jnp.minimum(dist_ref[...], d)
        dist_ref[...] = dmin
        m = jnp.max(dmin, axis=1, keepdims=True)
        idx = jnp.min(jnp.where(dmin == m, iota, _N), axis=1, keepdims=True)
        sel = iota == idx
        ncx = jnp.sum(jnp.where(sel, X, 0.0), axis=1, keepdims=True)
        ncy = jnp.sum(jnp.where(sel, Y, 0.0), axis=1, keepdims=True)
        ncz = jnp.sum(jnp.where(sel, Z, 0.0), axis=1, keepdims=True)
        return (ncx, ncy, ncz)

    c0 = (X[:, 0:1], Y[:, 0:1], Z[:, 0:1])
    jax.lax.fori_loop(0, _NPOINT, body, c0)


def _fps_new_xyz(xyz):
    X = xyz[:, :, 0]
    Y = xyz[:, :, 1]
    Z = xyz[:, :, 2]
    out_shape = [jax.ShapeDtypeStruct((_NPOINT, _B), jnp.float32)] * 3
    cx, cy, cz = pl.pallas_call(
        _fps_kernel,
        out_shape=out_shape,
        scratch_shapes=[pltpu.VMEM((_B, _N), jnp.float32)],
    )(X, Y, Z)
    return jnp.stack([cx.T, cy.T, cz.T], axis=-1)


def kernel(xyz, features, W0, g0, b0, W1, g1, b1, W2, g2, b2):
    new_xyz = _fps_new_xyz(xyz)

    s2 = jnp.sum(new_xyz ** 2, axis=-1)[:, :, None]
    d2 = jnp.sum(xyz ** 2, axis=-1)[:, None, :]
    cross = jnp.einsum('bnd,bmd->bnm', new_xyz, xyz)
    dist = jnp.maximum(s2 + d2 - 2.0 * cross, 0.0)
    _, group_idx = jax.lax.top_k(-dist, _K)

    grouped_xyz = jax.vmap(lambda p, i: p[i])(xyz, group_idx)
    grouped_xyz_norm = grouped_xyz - new_xyz[:, :, None, :]
    feats_t = jnp.transpose(features, (0, 2, 1))
    grouped_features = jax.vmap(lambda p, i: p[i])(feats_t, group_idx)
    grouped_features = jnp.transpose(grouped_features, (0, 3, 1, 2))
    grouped_input = jnp.concatenate(
        [jnp.transpose(grouped_xyz_norm, (0, 3, 1, 2)), grouped_features], axis=1
    )

    def mlp_layer(x, W, g, b):
        y = jnp.einsum('bcsk,oc->bosk', x, W)
        mean = jnp.mean(y, axis=(0, 2, 3), keepdims=True)
        var = jnp.var(y, axis=(0, 2, 3), keepdims=True)
        y = (y - mean) / jnp.sqrt(var + 1e-5)
        y = y * g[None, :, None, None] + b[None, :, None, None]
        return jax.nn.relu(y)

    y = mlp_layer(grouped_input, W0, g0, b0)
    y = mlp_layer(y, W1, g1, b1)
    y = mlp_layer(y, W2, g2, b2)
    new_features = jnp.max(y, axis=-1)
    return (new_xyz, new_features)


# speedup vs baseline: 2.5292x; 1.4474x over previous
import functools

import jax
import jax.numpy as jnp
from jax import lax
from jax.experimental import pallas as pl
from jax.experimental.pallas import tpu as pltpu
from jax.experimental.pallas import tpu_sc as plsc

_B = 4
_N = 8192
_S = 1024
_K = 32
_CIN = 32
_M = _B * _S * _K
_EPS = 1e-5


def _fps_kernel(x_ref, y_ref, z_ref, cx_ref, cy_ref, cz_ref, dist_ref):
    X = x_ref[...]
    Y = y_ref[...]
    Z = z_ref[...]
    dist_ref[...] = jnp.full((_B, _N), 1e10, dtype=jnp.float32)
    iota = lax.broadcasted_iota(jnp.int32, (_B, _N), 1)

    def body(i, carry):
        cx, cy, cz = carry
        cx_ref[pl.ds(i, 1), :] = cx.reshape(1, _B)
        cy_ref[pl.ds(i, 1), :] = cy.reshape(1, _B)
        cz_ref[pl.ds(i, 1), :] = cz.reshape(1, _B)
        dx = X - cx
        dy = Y - cy
        dz = Z - cz
        d = dx * dx + dy * dy + dz * dz
        dmin = jnp.minimum(dist_ref[...], d)
        dist_ref[...] = dmin
        m = jnp.max(dmin, axis=1, keepdims=True)
        idx = jnp.min(jnp.where(dmin == m, iota, _N), axis=1, keepdims=True)
        sel = iota == idx
        ncx = jnp.sum(jnp.where(sel, X, 0.0), axis=1, keepdims=True)
        ncy = jnp.sum(jnp.where(sel, Y, 0.0), axis=1, keepdims=True)
        ncz = jnp.sum(jnp.where(sel, Z, 0.0), axis=1, keepdims=True)
        return (ncx, ncy, ncz)

    c0 = (X[:, 0:1], Y[:, 0:1], Z[:, 0:1])
    lax.fori_loop(0, _S, body, c0)


def _fps_new_xyz(xyz):
    X = xyz[:, :, 0]
    Y = xyz[:, :, 1]
    Z = xyz[:, :, 2]
    out_shape = [jax.ShapeDtypeStruct((_S, _B), jnp.float32)] * 3
    cx, cy, cz = pl.pallas_call(
        _fps_kernel,
        out_shape=out_shape,
        scratch_shapes=[pltpu.VMEM((_B, _N), jnp.float32)],
    )(X, Y, Z)
    return jnp.stack([cx.T, cy.T, cz.T], axis=-1)


def _proj_kernel(xyz_ref, feats_ref, nxyz_ref, wxyz_ref, wfeat_ref, p_ref, q_ref):
    xyz = xyz_ref[0]
    feats = feats_ref[0]
    nxyz = nxyz_ref[0]
    wxyz = wxyz_ref[...]
    wfeat = wfeat_ref[...]
    p1 = lax.dot_general(xyz, wxyz, (((1,), (1,)), ((), ())),
                         preferred_element_type=jnp.float32)
    p2 = lax.dot_general(feats, wfeat, (((0,), (1,)), ((), ())),
                         preferred_element_type=jnp.float32)
    p_ref[0] = jnp.concatenate(
        [p1 + p2, jnp.zeros((_N, 96), jnp.float32)], axis=1)
    q_ref[0] = lax.dot_general(nxyz, wxyz, (((1,), (1,)), ((), ())),
                               preferred_element_type=jnp.float32)


def _project(xyz, features, new_xyz, W0):
    wxyz = W0[:, 0:3]
    wfeat = W0[:, 3:]
    p, q = pl.pallas_call(
        _proj_kernel,
        grid=(_B,),
        in_specs=[
            pl.BlockSpec((1, _N, 3), lambda b: (b, 0, 0)),
            pl.BlockSpec((1, _CIN, _N), lambda b: (b, 0, 0)),
            pl.BlockSpec((1, _S, 3), lambda b: (b, 0, 0)),
            pl.BlockSpec((32, 3), lambda b: (0, 0)),
            pl.BlockSpec((32, _CIN), lambda b: (0, 0)),
        ],
        out_specs=[
            pl.BlockSpec((1, _N, 128), lambda b: (b, 0, 0)),
            pl.BlockSpec((1, _S, 32), lambda b: (b, 0, 0)),
        ],
        out_shape=[
            jax.ShapeDtypeStruct((_B, _N, 128), jnp.float32),
            jax.ShapeDtypeStruct((_B, _S, 32), jnp.float32),
        ],
    )(xyz, features, new_xyz, wxyz, wfeat)
    return p, q


_NW = 32
_ROWS_PER_W = _M // _NW
_CHUNK = 128
_NCHUNK = _ROWS_PER_W // _CHUNK


def _sc_gather(p_flat, flat_idx):
    mesh = plsc.VectorSubcoreMesh(core_axis_name="c", subcore_axis_name="s")

    @functools.partial(
        pl.kernel,
        mesh=mesh,
        out_type=jax.ShapeDtypeStruct((_M, 128), jnp.float32),
        scratch_types=[
            pltpu.VMEM((_CHUNK,), jnp.int32),
            pltpu.VMEM((_CHUNK, 128), jnp.float32),
            pltpu.SemaphoreType.DMA,
        ],
    )
    def k(table_hbm, idx_hbm, out_hbm, idx_v, rows_v, sem):
        wid = lax.axis_index("s") * 2 + lax.axis_index("c")
        base = wid * _ROWS_PER_W

        def body(j, carry):
            off = base + j * _CHUNK
            pltpu.sync_copy(idx_hbm.at[pl.ds(off, _CHUNK)], idx_v)
            pltpu.async_copy(table_hbm.at[idx_v], rows_v, sem).wait()
            pltpu.sync_copy(rows_v, out_hbm.at[pl.ds(off, _CHUNK)])
            return carry

        lax.fori_loop(0, _NCHUNK, body, 0)

    return k(p_flat, flat_idx)


_RCHUNK = 4096
_NSTEP = _M // _RCHUNK
_SCH = _RCHUNK // _K


def _passA_kernel(g_ref, q_ref, y_ref, st_ref, acc_ref):
    step = pl.program_id(0)

    @pl.when(step == 0)
    def _():
        acc_ref[...] = jnp.zeros_like(acc_ref)

    g = g_ref[:, 0:32]
    q = q_ref[0]
    qrep = jnp.broadcast_to(q[:, None, :], (_SCH, _K, 32)).reshape(_RCHUNK, 32)
    y = g - qrep
    y_ref[...] = y
    acc_ref[0:1, 0:32] += jnp.sum(y, axis=0, keepdims=True)
    acc_ref[1:2, 0:32] += jnp.sum(y * y, axis=0, keepdims=True)

    @pl.when(step == _NSTEP - 1)
    def _():
        st_ref[...] = acc_ref[...]


def _passA(g, q):
    return pl.pallas_call(
        _passA_kernel,
        grid=(_NSTEP,),
        in_specs=[
            pl.BlockSpec((_RCHUNK, 128), lambda i: (i, 0)),
            pl.BlockSpec((1, _SCH, 32), lambda i: (i, 0, 0)),
        ],
        out_specs=[
            pl.BlockSpec((_RCHUNK, 32), lambda i: (i, 0)),
            pl.BlockSpec((8, 128), lambda i: (0, 0)),
        ],
        out_shape=[
            jax.ShapeDtypeStruct((_M, 32), jnp.float32),
            jax.ShapeDtypeStruct((8, 128), jnp.float32),
        ],
        scratch_shapes=[pltpu.VMEM((8, 128), jnp.float32)],
    )(g, q.reshape(_NSTEP, _SCH, 32))


def _norm_relu(y, st_ref, gam_ref, bet_ref, cdim):
    s = st_ref[0:1, 0:cdim]
    ss = st_ref[1:2, 0:cdim]
    mean = s * (1.0 / _M)
    var = ss * (1.0 / _M) - mean * mean
    inv = lax.rsqrt(var + _EPS)
    gam = gam_ref[...]
    bet = bet_ref[...]
    return jnp.maximum((y - mean) * inv * gam + bet, 0.0)


def _passBC_kernel(y_ref, st_ref, gam_ref, bet_ref, w_ref, o_ref, so_ref,
                   acc_ref, *, cin, cout):
    step = pl.program_id(0)

    @pl.when(step == 0)
    def _():
        acc_ref[...] = jnp.zeros_like(acc_ref)

    x = _norm_relu(y_ref[...], st_ref, gam_ref, bet_ref, cin)
    ynew = lax.dot_general(x, w_ref[...], (((1,), (1,)), ((), ())),
                           preferred_element_type=jnp.float32)
    o_ref[...] = ynew
    acc_ref[0:1, 0:cout] += jnp.sum(ynew, axis=0, keepdims=True)
    acc_ref[1:2, 0:cout] += jnp.sum(ynew * ynew, axis=0, keepdims=True)

    @pl.when(step == _NSTEP - 1)
    def _():
        so_ref[...] = acc_ref[...]


def _passBC(y, st, gam, bet, w, cin, cout):
    return pl.pallas_call(
        functools.partial(_passBC_kernel, cin=cin, cout=cout),
        grid=(_NSTEP,),
        in_specs=[
            pl.BlockSpec((_RCHUNK, cin), lambda i: (i, 0)),
            pl.BlockSpec((8, 128), lambda i: (0, 0)),
            pl.BlockSpec((1, cin), lambda i: (0, 0)),
            pl.BlockSpec((1, cin), lambda i: (0, 0)),
            pl.BlockSpec((cout, cin), lambda i: (0, 0)),
        ],
        out_specs=[
            pl.BlockSpec((_RCHUNK, cout), lambda i: (i, 0)),
            pl.BlockSpec((8, 128), lambda i: (0, 0)),
        ],
        out_shape=[
            jax.ShapeDtypeStruct((_M, cout), jnp.float32),
            jax.ShapeDtypeStruct((8, 128), jnp.float32),
        ],
        scratch_shapes=[pltpu.VMEM((8, 128), jnp.float32)],
    )(y, st, gam.reshape(1, cin), bet.reshape(1, cin), w)


def _passD_kernel(y_ref, st_ref, gam_ref, bet_ref, o_ref):
    x = _norm_relu(y_ref[...], st_ref, gam_ref, bet_ref, 64)
    x3 = x.reshape(_SCH, _K, 64)
    o_ref[...] = jnp.max(x3, axis=1)


def _passD(y, st, gam, bet):
    return pl.pallas_call(
        _passD_kernel,
        grid=(_NSTEP,),
        in_specs=[
            pl.BlockSpec((_RCHUNK, 64), lambda i: (i, 0)),
            pl.BlockSpec((8, 128), lambda i: (0, 0)),
            pl.BlockSpec((1, 64), lambda i: (0, 0)),
            pl.BlockSpec((1, 64), lambda i: (0, 0)),
        ],
        out_specs=pl.BlockSpec((_SCH, 64), lambda i: (i, 0)),
        out_shape=jax.ShapeDtypeStruct((_B * _S, 64), jnp.float32),
    )(y, st, gam.reshape(1, 64), bet.reshape(1, 64))


def kernel(xyz, features, W0, g0, b0, W1, g1, b1, W2, g2, b2):
    new_xyz = _fps_new_xyz(xyz)

    s2 = jnp.sum(new_xyz ** 2, axis=-1)[:, :, None]
    d2 = jnp.sum(xyz ** 2, axis=-1)[:, None, :]
    cross = jnp.einsum('bnd,bmd->bnm', new_xyz, xyz)
    dist = jnp.maximum(s2 + d2 - 2.0 * cross, 0.0)
    _, group_idx = lax.top_k(-dist, _K)

    p, q = _project(xyz, features, new_xyz, W0)
    flat_idx = (group_idx
                + jnp.arange(_B, dtype=jnp.int32)[:, None, None] * _N
                ).reshape(_M)
    g = _sc_gather(p.reshape(_B * _N, 128), flat_idx)

    y1, st1 = _passA(g, q)
    y2, st2 = _passBC(y1, st1, g0, b0, W1, 32, 32)
    y3, st3 = _passBC(y2, st2, g1, b1, W2, 32, 64)
    nf = _passD(y3, st3, g2, b2)

    new_features = nf.reshape(_B, _S, 64).transpose(0, 2, 1)
    return (new_xyz, new_features)


# speedup vs baseline: 10.3511x; 4.0926x over previous
import functools

import jax
import jax.numpy as jnp
from jax import lax
from jax.experimental import pallas as pl
from jax.experimental.pallas import tpu as pltpu
from jax.experimental.pallas import tpu_sc as plsc

_B = 4
_N = 8192
_S = 1024
_K = 32
_CIN = 32
_M = _B * _S * _K
_EPS = 1e-5


def _fps_kernel(x_ref, y_ref, z_ref, cx_ref, cy_ref, cz_ref, dist_ref):
    X = x_ref[...]
    Y = y_ref[...]
    Z = z_ref[...]
    dist_ref[...] = jnp.full((_B, _N), 1e10, dtype=jnp.float32)
    iota = lax.broadcasted_iota(jnp.int32, (_B, _N), 1)

    def body(i, carry):
        cx, cy, cz = carry
        cx_ref[pl.ds(i, 1), :] = cx.reshape(1, _B)
        cy_ref[pl.ds(i, 1), :] = cy.reshape(1, _B)
        cz_ref[pl.ds(i, 1), :] = cz.reshape(1, _B)
        dx = X - cx
        dy = Y - cy
        dz = Z - cz
        d = dx * dx + dy * dy + dz * dz
        dmin = jnp.minimum(dist_ref[...], d)
        dist_ref[...] = dmin
        m = jnp.max(dmin, axis=1, keepdims=True)
        idx = jnp.min(jnp.where(dmin == m, iota, _N), axis=1, keepdims=True)
        sel = iota == idx
        ncx = jnp.sum(jnp.where(sel, X, 0.0), axis=1, keepdims=True)
        ncy = jnp.sum(jnp.where(sel, Y, 0.0), axis=1, keepdims=True)
        ncz = jnp.sum(jnp.where(sel, Z, 0.0), axis=1, keepdims=True)
        return (ncx, ncy, ncz)

    c0 = (X[:, 0:1], Y[:, 0:1], Z[:, 0:1])
    lax.fori_loop(0, _S, body, c0)


def _fps_new_xyz(xyz):
    X = xyz[:, :, 0]
    Y = xyz[:, :, 1]
    Z = xyz[:, :, 2]
    out_shape = [jax.ShapeDtypeStruct((_S, _B), jnp.float32)] * 3
    cx, cy, cz = pl.pallas_call(
        _fps_kernel,
        out_shape=out_shape,
        scratch_shapes=[pltpu.VMEM((_B, _N), jnp.float32)],
    )(X, Y, Z)
    return jnp.stack([cx.T, cy.T, cz.T], axis=-1)


def _proj_kernel(xyz_ref, feats_ref, nxyz_ref, wxyz_ref, wfeat_ref, p_ref, q_ref):
    xyz = xyz_ref[0]
    feats = feats_ref[0]
    nxyz = nxyz_ref[0]
    wxyz = wxyz_ref[...]
    wfeat = wfeat_ref[...]
    p1 = lax.dot_general(xyz, wxyz, (((1,), (1,)), ((), ())),
                         preferred_element_type=jnp.float32)
    p2 = lax.dot_general(feats, wfeat, (((0,), (1,)), ((), ())),
                         preferred_element_type=jnp.float32)
    p_ref[0] = jnp.concatenate(
        [p1 + p2, jnp.zeros((_N, 96), jnp.float32)], axis=1)
    q_ref[0] = lax.dot_general(nxyz, wxyz, (((1,), (1,)), ((), ())),
                               preferred_element_type=jnp.float32)


def _project(xyz, features, new_xyz, W0):
    wxyz = W0[:, 0:3]
    wfeat = W0[:, 3:]
    p, q = pl.pallas_call(
        _proj_kernel,
        grid=(_B,),
        in_specs=[
            pl.BlockSpec((1, _N, 3), lambda b: (b, 0, 0)),
            pl.BlockSpec((1, _CIN, _N), lambda b: (b, 0, 0)),
            pl.BlockSpec((1, _S, 3), lambda b: (b, 0, 0)),
            pl.BlockSpec((32, 3), lambda b: (0, 0)),
            pl.BlockSpec((32, _CIN), lambda b: (0, 0)),
        ],
        out_specs=[
            pl.BlockSpec((1, _N, 128), lambda b: (b, 0, 0)),
            pl.BlockSpec((1, _S, 32), lambda b: (b, 0, 0)),
        ],
        out_shape=[
            jax.ShapeDtypeStruct((_B, _N, 128), jnp.float32),
            jax.ShapeDtypeStruct((_B, _S, 32), jnp.float32),
        ],
    )(xyz, features, new_xyz, wxyz, wfeat)
    return p, q


_KTILE = 128


def _knn_kernel(nxyz_ref, xyz_ref, idx_ref, d_ref, io_ref):
    c = nxyz_ref[0]
    x = xyz_ref[0]
    s2 = jnp.sum(c * c, axis=1, keepdims=True)
    d2 = jnp.sum(x * x, axis=1, keepdims=True)
    cross = lax.dot_general(c, x, (((1,), (1,)), ((), ())))
    d_ref[...] = jnp.maximum(s2 + d2[:, 0][None, :] - 2.0 * cross, 0.0)
    io_ref[...] = lax.broadcasted_iota(jnp.int32, (_KTILE, _N), 1)

    kiota = lax.broadcasted_iota(jnp.int32, (_KTILE, _K), 1)

    def body(j, acc):
        d = d_ref[...]
        iota = io_ref[...]
        m = jnp.min(d, axis=1, keepdims=True)
        idx = jnp.min(jnp.where(d == m, iota, _N), axis=1, keepdims=True)
        d_ref[...] = jnp.where(iota == idx, jnp.float32(3e38), d)
        return jnp.where(kiota == j, idx, acc)

    idx_ref[0] = lax.fori_loop(
        0, _K, body, jnp.zeros((_KTILE, _K), jnp.int32))


def _knn(new_xyz, xyz):
    return pl.pallas_call(
        _knn_kernel,
        grid=(_B, _S // _KTILE),
        in_specs=[
            pl.BlockSpec((1, _KTILE, 3), lambda b, s: (b, s, 0)),
            pl.BlockSpec((1, _N, 3), lambda b, s: (b, 0, 0)),
        ],
        out_specs=pl.BlockSpec((1, _KTILE, _K), lambda b, s: (b, s, 0)),
        out_shape=jax.ShapeDtypeStruct((_B, _S, _K), jnp.int32),
        scratch_shapes=[
            pltpu.VMEM((_KTILE, _N), jnp.float32),
            pltpu.VMEM((_KTILE, _N), jnp.int32),
        ],
    )(new_xyz, xyz)


_NW = 32
_ROWS_PER_W = _M // _NW
_CHUNK = 128
_NCHUNK = _ROWS_PER_W // _CHUNK


def _sc_gather(p_flat, flat_idx):
    mesh = plsc.VectorSubcoreMesh(core_axis_name="c", subcore_axis_name="s")

    @functools.partial(
        pl.kernel,
        mesh=mesh,
        out_type=jax.ShapeDtypeStruct((_M, 128), jnp.float32),
        scratch_types=[
            pltpu.VMEM((_CHUNK,), jnp.int32),
            pltpu.VMEM((_CHUNK, 128), jnp.float32),
            pltpu.SemaphoreType.DMA,
        ],
    )
    def k(table_hbm, idx_hbm, out_hbm, idx_v, rows_v, sem):
        wid = lax.axis_index("s") * 2 + lax.axis_index("c")
        base = wid * _ROWS_PER_W

        def body(j, carry):
            off = base + j * _CHUNK
            pltpu.sync_copy(idx_hbm.at[pl.ds(off, _CHUNK)], idx_v)
            pltpu.async_copy(table_hbm.at[idx_v], rows_v, sem).wait()
            pltpu.sync_copy(rows_v, out_hbm.at[pl.ds(off, _CHUNK)])
            return carry

        lax.fori_loop(0, _NCHUNK, body, 0)

    return k(p_flat, flat_idx)


_RCHUNK = 4096
_NSTEP = _M // _RCHUNK
_SCH = _RCHUNK // _K


def _passA_kernel(g_ref, q_ref, y_ref, st_ref, acc_ref):
    step = pl.program_id(0)

    @pl.when(step == 0)
    def _():
        acc_ref[...] = jnp.zeros_like(acc_ref)

    g = g_ref[:, 0:32]
    q = q_ref[0]
    qrep = jnp.broadcast_to(q[:, None, :], (_SCH, _K, 32)).reshape(_RCHUNK, 32)
    y = g - qrep
    y_ref[...] = y
    acc_ref[0:1, 0:32] += jnp.sum(y, axis=0, keepdims=True)
    acc_ref[1:2, 0:32] += jnp.sum(y * y, axis=0, keepdims=True)

    @pl.when(step == _NSTEP - 1)
    def _():
        st_ref[...] = acc_ref[...]


def _passA(g, q):
    return pl.pallas_call(
        _passA_kernel,
        grid=(_NSTEP,),
        in_specs=[
            pl.BlockSpec((_RCHUNK, 128), lambda i: (i, 0)),
            pl.BlockSpec((1, _SCH, 32), lambda i: (i, 0, 0)),
        ],
        out_specs=[
            pl.BlockSpec((_RCHUNK, 32), lambda i: (i, 0)),
            pl.BlockSpec((8, 128), lambda i: (0, 0)),
        ],
        out_shape=[
            jax.ShapeDtypeStruct((_M, 32), jnp.float32),
            jax.ShapeDtypeStruct((8, 128), jnp.float32),
        ],
        scratch_shapes=[pltpu.VMEM((8, 128), jnp.float32)],
    )(g, q.reshape(_NSTEP, _SCH, 32))


def _norm_relu(y, st_ref, gam_ref, bet_ref, cdim):
    s = st_ref[0:1, 0:cdim]
    ss = st_ref[1:2, 0:cdim]
    mean = s * (1.0 / _M)
    var = ss * (1.0 / _M) - mean * mean
    inv = lax.rsqrt(var + _EPS)
    gam = gam_ref[...]
    bet = bet_ref[...]
    return jnp.maximum((y - mean) * inv * gam + bet, 0.0)


def _passBC_kernel(y_ref, st_ref, gam_ref, bet_ref, w_ref, o_ref, so_ref,
                   acc_ref, *, cin, cout):
    step = pl.program_id(0)

    @pl.when(step == 0)
    def _():
        acc_ref[...] = jnp.zeros_like(acc_ref)

    x = _norm_relu(y_ref[...], st_ref, gam_ref, bet_ref, cin)
    ynew = lax.dot_general(x, w_ref[...], (((1,), (1,)), ((), ())),
                           preferred_element_type=jnp.float32)
    o_ref[...] = ynew
    acc_ref[0:1, 0:cout] += jnp.sum(ynew, axis=0, keepdims=True)
    acc_ref[1:2, 0:cout] += jnp.sum(ynew * ynew, axis=0, keepdims=True)

    @pl.when(step == _NSTEP - 1)
    def _():
        so_ref[...] = acc_ref[...]


def _passBC(y, st, gam, bet, w, cin, cout):
    return pl.pallas_call(
        functools.partial(_passBC_kernel, cin=cin, cout=cout),
        grid=(_NSTEP,),
        in_specs=[
            pl.BlockSpec((_RCHUNK, cin), lambda i: (i, 0)),
            pl.BlockSpec((8, 128), lambda i: (0, 0)),
            pl.BlockSpec((1, cin), lambda i: (0, 0)),
            pl.BlockSpec((1, cin), lambda i: (0, 0)),
            pl.BlockSpec((cout, cin), lambda i: (0, 0)),
        ],
        out_specs=[
            pl.BlockSpec((_RCHUNK, cout), lambda i: (i, 0)),
            pl.BlockSpec((8, 128), lambda i: (0, 0)),
        ],
        out_shape=[
            jax.ShapeDtypeStruct((_M, cout), jnp.float32),
            jax.ShapeDtypeStruct((8, 128), jnp.float32),
        ],
        scratch_shapes=[pltpu.VMEM((8, 128), jnp.float32)],
    )(y, st, gam.reshape(1, cin), bet.reshape(1, cin), w)


def _passD_kernel(y_ref, st_ref, gam_ref, bet_ref, o_ref):
    x = _norm_relu(y_ref[...], st_ref, gam_ref, bet_ref, 64)
    x3 = x.reshape(_SCH, _K, 64)
    o_ref[...] = jnp.max(x3, axis=1)


def _passD(y, st, gam, bet):
    return pl.pallas_call(
        _passD_kernel,
        grid=(_NSTEP,),
        in_specs=[
            pl.BlockSpec((_RCHUNK, 64), lambda i: (i, 0)),
            pl.BlockSpec((8, 128), lambda i: (0, 0)),
            pl.BlockSpec((1, 64), lambda i: (0, 0)),
            pl.BlockSpec((1, 64), lambda i: (0, 0)),
        ],
        out_specs=pl.BlockSpec((_SCH, 64), lambda i: (i, 0)),
        out_shape=jax.ShapeDtypeStruct((_B * _S, 64), jnp.float32),
    )(y, st, gam.reshape(1, 64), bet.reshape(1, 64))


def kernel(xyz, features, W0, g0, b0, W1, g1, b1, W2, g2, b2):
    new_xyz = _fps_new_xyz(xyz)

    group_idx = _knn(new_xyz, xyz)

    p, q = _project(xyz, features, new_xyz, W0)
    flat_idx = (group_idx
                + jnp.arange(_B, dtype=jnp.int32)[:, None, None] * _N
                ).reshape(_M)
    g = _sc_gather(p.reshape(_B * _N, 128), flat_idx)

    y1, st1 = _passA(g, q)
    y2, st2 = _passBC(y1, st1, g0, b0, W1, 32, 32)
    y3, st3 = _passBC(y2, st2, g1, b1, W2, 32, 64)
    nf = _passD(y3, st3, g2, b2)

    new_features = nf.reshape(_B, _S, 64).transpose(0, 2, 1)
    return (new_xyz, new_features)


# speedup vs baseline: 10.9885x; 1.0616x over previous
import functools

import jax
import jax.numpy as jnp
from jax import lax
from jax.experimental import pallas as pl
from jax.experimental.pallas import tpu as pltpu
from jax.experimental.pallas import tpu_sc as plsc

_B = 4
_N = 8192
_S = 1024
_K = 32
_CIN = 32
_M = _B * _S * _K
_EPS = 1e-5


_FR = _N // 128


def _fps_kernel(x_ref, y_ref, z_ref, cx_ref, cy_ref, cz_ref, dist_ref, io_ref):
    sh = (_B, _FR, 128)
    dist_ref[...] = jnp.full((_B * _FR, 128), 1e10, dtype=jnp.float32)
    io_ref[...] = (
        lax.broadcasted_iota(jnp.int32, sh, 1) * 128
        + lax.broadcasted_iota(jnp.int32, sh, 2)
    ).reshape(_B * _FR, 128)

    def body(i, carry):
        cx, cy, cz = carry
        cx_ref[pl.ds(i, 1), :] = cx.reshape(1, _B)
        cy_ref[pl.ds(i, 1), :] = cy.reshape(1, _B)
        cz_ref[pl.ds(i, 1), :] = cz.reshape(1, _B)
        X = x_ref[...].reshape(sh)
        Y = y_ref[...].reshape(sh)
        Z = z_ref[...].reshape(sh)
        dx = X - cx
        dy = Y - cy
        dz = Z - cz
        d = dx * dx + dy * dy + dz * dz
        dmin = jnp.minimum(dist_ref[...].reshape(sh), d)
        dist_ref[...] = dmin.reshape(_B * _FR, 128)
        io = io_ref[...].reshape(sh)
        m = jnp.max(dmin, axis=(1, 2), keepdims=True)
        idx = jnp.min(jnp.where(dmin == m, io, _N), axis=(1, 2), keepdims=True)
        sel = io == idx
        ncx = jnp.sum(jnp.where(sel, X, 0.0), axis=(1, 2), keepdims=True)
        ncy = jnp.sum(jnp.where(sel, Y, 0.0), axis=(1, 2), keepdims=True)
        ncz = jnp.sum(jnp.where(sel, Z, 0.0), axis=(1, 2), keepdims=True)
        return (ncx, ncy, ncz)

    X0 = x_ref[...].reshape(sh)
    Y0 = y_ref[...].reshape(sh)
    Z0 = z_ref[...].reshape(sh)
    c0 = (X0[:, 0:1, 0:1], Y0[:, 0:1, 0:1], Z0[:, 0:1, 0:1])
    lax.fori_loop(0, _S, body, c0)


def _fps_new_xyz(xyz):
    X = xyz[:, :, 0].reshape(_B * _FR, 128)
    Y = xyz[:, :, 1].reshape(_B * _FR, 128)
    Z = xyz[:, :, 2].reshape(_B * _FR, 128)
    out_shape = [jax.ShapeDtypeStruct((_S, _B), jnp.float32)] * 3
    cx, cy, cz = pl.pallas_call(
        _fps_kernel,
        out_shape=out_shape,
        scratch_shapes=[
            pltpu.VMEM((_B * _FR, 128), jnp.float32),
            pltpu.VMEM((_B * _FR, 128), jnp.int32),
        ],
    )(X, Y, Z)
    return jnp.stack([cx.T, cy.T, cz.T], axis=-1)


def _proj_kernel(xyz_ref, feats_ref, nxyz_ref, wxyz_ref, wfeat_ref, p_ref, q_ref):
    xyz = xyz_ref[0]
    feats = feats_ref[0]
    nxyz = nxyz_ref[0]
    wxyz = wxyz_ref[...]
    wfeat = wfeat_ref[...]
    p1 = lax.dot_general(xyz, wxyz, (((1,), (1,)), ((), ())),
                         preferred_element_type=jnp.float32)
    p2 = lax.dot_general(feats, wfeat, (((0,), (1,)), ((), ())),
                         preferred_element_type=jnp.float32)
    p_ref[0] = jnp.concatenate(
        [p1 + p2, jnp.zeros((_N, 96), jnp.float32)], axis=1)
    q_ref[0] = lax.dot_general(nxyz, wxyz, (((1,), (1,)), ((), ())),
                               preferred_element_type=jnp.float32)


def _project(xyz, features, new_xyz, W0):
    wxyz = W0[:, 0:3]
    wfeat = W0[:, 3:]
    p, q = pl.pallas_call(
        _proj_kernel,
        grid=(_B,),
        in_specs=[
            pl.BlockSpec((1, _N, 3), lambda b: (b, 0, 0)),
            pl.BlockSpec((1, _CIN, _N), lambda b: (b, 0, 0)),
            pl.BlockSpec((1, _S, 3), lambda b: (b, 0, 0)),
            pl.BlockSpec((32, 3), lambda b: (0, 0)),
            pl.BlockSpec((32, _CIN), lambda b: (0, 0)),
        ],
        out_specs=[
            pl.BlockSpec((1, _N, 128), lambda b: (b, 0, 0)),
            pl.BlockSpec((1, _S, 32), lambda b: (b, 0, 0)),
        ],
        out_shape=[
            jax.ShapeDtypeStruct((_B, _N, 128), jnp.float32),
            jax.ShapeDtypeStruct((_B, _S, 32), jnp.float32),
        ],
    )(xyz, features, new_xyz, wxyz, wfeat)
    return p, q


_KTILE = 128


def _knn_kernel(nxyz_ref, xyz_ref, idx_ref, d_ref, io_ref):
    c = nxyz_ref[0]
    x = xyz_ref[0]
    s2 = jnp.sum(c * c, axis=1, keepdims=True)
    d2 = jnp.sum(x * x, axis=1, keepdims=True)
    cross = lax.dot_general(c, x, (((1,), (1,)), ((), ())))
    d_ref[...] = jnp.maximum(s2 + d2[:, 0][None, :] - 2.0 * cross, 0.0)
    io_ref[...] = lax.broadcasted_iota(jnp.int32, (_KTILE, _N), 1)

    kiota = lax.broadcasted_iota(jnp.int32, (_KTILE, _K), 1)

    def body(j, acc):
        d = d_ref[...]
        iota = io_ref[...]
        m = jnp.min(d, axis=1, keepdims=True)
        idx = jnp.min(jnp.where(d == m, iota, _N), axis=1, keepdims=True)
        d_ref[...] = jnp.where(iota == idx, jnp.float32(3e38), d)
        return jnp.where(kiota == j, idx, acc)

    idx_ref[0] = lax.fori_loop(
        0, _K, body, jnp.zeros((_KTILE, _K), jnp.int32))


def _knn(new_xyz, xyz):
    return pl.pallas_call(
        _knn_kernel,
        grid=(_B, _S // _KTILE),
        in_specs=[
            pl.BlockSpec((1, _KTILE, 3), lambda b, s: (b, s, 0)),
            pl.BlockSpec((1, _N, 3), lambda b, s: (b, 0, 0)),
        ],
        out_specs=pl.BlockSpec((1, _KTILE, _K), lambda b, s: (b, s, 0)),
        out_shape=jax.ShapeDtypeStruct((_B, _S, _K), jnp.int32),
        scratch_shapes=[
            pltpu.VMEM((_KTILE, _N), jnp.float32),
            pltpu.VMEM((_KTILE, _N), jnp.int32),
        ],
    )(new_xyz, xyz)


_NW = 32
_ROWS_PER_W = _M // _NW
_CHUNK = 128
_NCHUNK = _ROWS_PER_W // _CHUNK


def _sc_gather(p_flat, flat_idx):
    mesh = plsc.VectorSubcoreMesh(core_axis_name="c", subcore_axis_name="s")

    @functools.partial(
        pl.kernel,
        mesh=mesh,
        out_type=jax.ShapeDtypeStruct((_M, 128), jnp.float32),
        scratch_types=[
            pltpu.VMEM((_CHUNK,), jnp.int32),
            pltpu.VMEM((_CHUNK, 128), jnp.float32),
            pltpu.SemaphoreType.DMA,
        ],
    )
    def k(table_hbm, idx_hbm, out_hbm, idx_v, rows_v, sem):
        wid = lax.axis_index("s") * 2 + lax.axis_index("c")
        base = wid * _ROWS_PER_W

        def body(j, carry):
            off = base + j * _CHUNK
            pltpu.sync_copy(idx_hbm.at[pl.ds(off, _CHUNK)], idx_v)
            pltpu.async_copy(table_hbm.at[idx_v], rows_v, sem).wait()
            pltpu.sync_copy(rows_v, out_hbm.at[pl.ds(off, _CHUNK)])
            return carry

        lax.fori_loop(0, _NCHUNK, body, 0)

    return k(p_flat, flat_idx)


_RCHUNK = 4096
_NSTEP = _M // _RCHUNK
_SCH = _RCHUNK // _K


def _passA_kernel(g_ref, q_ref, y_ref, st_ref, acc_ref):
    step = pl.program_id(0)

    @pl.when(step == 0)
    def _():
        acc_ref[...] = jnp.zeros_like(acc_ref)

    g = g_ref[:, 0:32]
    q = q_ref[0]
    qrep = jnp.broadcast_to(q[:, None, :], (_SCH, _K, 32)).reshape(_RCHUNK, 32)
    y = g - qrep
    y_ref[...] = y
    acc_ref[0:1, 0:32] += jnp.sum(y, axis=0, keepdims=True)
    acc_ref[1:2, 0:32] += jnp.sum(y * y, axis=0, keepdims=True)

    @pl.when(step == _NSTEP - 1)
    def _():
        st_ref[...] = acc_ref[...]


def _passA(g, q):
    return pl.pallas_call(
        _passA_kernel,
        grid=(_NSTEP,),
        in_specs=[
            pl.BlockSpec((_RCHUNK, 128), lambda i: (i, 0)),
            pl.BlockSpec((1, _SCH, 32), lambda i: (i, 0, 0)),
        ],
        out_specs=[
            pl.BlockSpec((_RCHUNK, 32), lambda i: (i, 0)),
            pl.BlockSpec((8, 128), lambda i: (0, 0)),
        ],
        out_shape=[
            jax.ShapeDtypeStruct((_M, 32), jnp.float32),
            jax.ShapeDtypeStruct((8, 128), jnp.float32),
        ],
        scratch_shapes=[pltpu.VMEM((8, 128), jnp.float32)],
    )(g, q.reshape(_NSTEP, _SCH, 32))


def _norm_relu(y, st_ref, gam_ref, bet_ref, cdim):
    s = st_ref[0:1, 0:cdim]
    ss = st_ref[1:2, 0:cdim]
    mean = s * (1.0 / _M)
    var = ss * (1.0 / _M) - mean * mean
    inv = lax.rsqrt(var + _EPS)
    gam = gam_ref[...]
    bet = bet_ref[...]
    return jnp.maximum((y - mean) * inv * gam + bet, 0.0)


def _passBC_kernel(y_ref, st_ref, gam_ref, bet_ref, w_ref, o_ref, so_ref,
                   acc_ref, *, cin, cout):
    step = pl.program_id(0)

    @pl.when(step == 0)
    def _():
        acc_ref[...] = jnp.zeros_like(acc_ref)

    x = _norm_relu(y_ref[...], st_ref, gam_ref, bet_ref, cin)
    ynew = lax.dot_general(x, w_ref[...], (((1,), (1,)), ((), ())),
                           preferred_element_type=jnp.float32)
    o_ref[...] = ynew
    acc_ref[0:1, 0:cout] += jnp.sum(ynew, axis=0, keepdims=True)
    acc_ref[1:2, 0:cout] += jnp.sum(ynew * ynew, axis=0, keepdims=True)

    @pl.when(step == _NSTEP - 1)
    def _():
        so_ref[...] = acc_ref[...]


def _passBC(y, st, gam, bet, w, cin, cout):
    return pl.pallas_call(
        functools.partial(_passBC_kernel, cin=cin, cout=cout),
        grid=(_NSTEP,),
        in_specs=[
            pl.BlockSpec((_RCHUNK, cin), lambda i: (i, 0)),
            pl.BlockSpec((8, 128), lambda i: (0, 0)),
            pl.BlockSpec((1, cin), lambda i: (0, 0)),
            pl.BlockSpec((1, cin), lambda i: (0, 0)),
            pl.BlockSpec((cout, cin), lambda i: (0, 0)),
        ],
        out_specs=[
            pl.BlockSpec((_RCHUNK, cout), lambda i: (i, 0)),
            pl.BlockSpec((8, 128), lambda i: (0, 0)),
        ],
        out_shape=[
            jax.ShapeDtypeStruct((_M, cout), jnp.float32),
            jax.ShapeDtypeStruct((8, 128), jnp.float32),
        ],
        scratch_shapes=[pltpu.VMEM((8, 128), jnp.float32)],
    )(y, st, gam.reshape(1, cin), bet.reshape(1, cin), w)


def _passD_kernel(y_ref, st_ref, gam_ref, bet_ref, o_ref):
    x = _norm_relu(y_ref[...], st_ref, gam_ref, bet_ref, 64)
    x3 = x.reshape(_SCH, _K, 64)
    o_ref[...] = jnp.max(x3, axis=1)


def _passD(y, st, gam, bet):
    return pl.pallas_call(
        _passD_kernel,
        grid=(_NSTEP,),
        in_specs=[
            pl.BlockSpec((_RCHUNK, 64), lambda i: (i, 0)),
            pl.BlockSpec((8, 128), lambda i: (0, 0)),
            pl.BlockSpec((1, 64), lambda i: (0, 0)),
            pl.BlockSpec((1, 64), lambda i: (0, 0)),
        ],
        out_specs=pl.BlockSpec((_SCH, 64), lambda i: (i, 0)),
        out_shape=jax.ShapeDtypeStruct((_B * _S, 64), jnp.float32),
    )(y, st, gam.reshape(1, 64), bet.reshape(1, 64))


def kernel(xyz, features, W0, g0, b0, W1, g1, b1, W2, g2, b2):
    new_xyz = _fps_new_xyz(xyz)

    group_idx = _knn(new_xyz, xyz)

    p, q = _project(xyz, features, new_xyz, W0)
    flat_idx = (group_idx
                + jnp.arange(_B, dtype=jnp.int32)[:, None, None] * _N
                ).reshape(_M)
    g = _sc_gather(p.reshape(_B * _N, 128), flat_idx)

    y1, st1 = _passA(g, q)
    y2, st2 = _passBC(y1, st1, g0, b0, W1, 32, 32)
    y3, st3 = _passBC(y2, st2, g1, b1, W2, 32, 64)
    nf = _passD(y3, st3, g2, b2)

    new_features = nf.reshape(_B, _S, 64).transpose(0, 2, 1)
    return (new_xyz, new_features)


# speedup vs baseline: 11.1213x; 1.0121x over previous
import functools

import jax
import jax.numpy as jnp
from jax import lax
from jax.experimental import pallas as pl
from jax.experimental.pallas import tpu as pltpu
from jax.experimental.pallas import tpu_sc as plsc

_B = 4
_N = 8192
_S = 1024
_K = 32
_CIN = 32
_M = _B * _S * _K
_EPS = 1e-5


_FR = _N // 128


def _fps_kernel(x_ref, y_ref, z_ref, cx_ref, cy_ref, cz_ref, dist_ref, io_ref):
    sh = (_B, _FR, 128)
    dist_ref[...] = jnp.full((_B * _FR, 128), 1e10, dtype=jnp.float32)
    io_ref[...] = (
        lax.broadcasted_iota(jnp.int32, sh, 1) * 128
        + lax.broadcasted_iota(jnp.int32, sh, 2)
    ).reshape(_B * _FR, 128)

    def body(i, carry):
        cx, cy, cz = carry
        cx_ref[pl.ds(i, 1), :] = cx.reshape(1, _B)
        cy_ref[pl.ds(i, 1), :] = cy.reshape(1, _B)
        cz_ref[pl.ds(i, 1), :] = cz.reshape(1, _B)
        X = x_ref[...].reshape(sh)
        Y = y_ref[...].reshape(sh)
        Z = z_ref[...].reshape(sh)
        dx = X - cx
        dy = Y - cy
        dz = Z - cz
        d = dx * dx + dy * dy + dz * dz
        dmin = jnp.minimum(dist_ref[...].reshape(sh), d)
        dist_ref[...] = dmin.reshape(_B * _FR, 128)
        io = io_ref[...].reshape(sh)
        m = jnp.max(dmin, axis=(1, 2), keepdims=True)
        idx = jnp.min(jnp.where(dmin == m, io, _N), axis=(1, 2), keepdims=True)
        sel = io == idx
        ncx = jnp.sum(jnp.where(sel, X, 0.0), axis=(1, 2), keepdims=True)
        ncy = jnp.sum(jnp.where(sel, Y, 0.0), axis=(1, 2), keepdims=True)
        ncz = jnp.sum(jnp.where(sel, Z, 0.0), axis=(1, 2), keepdims=True)
        return (ncx, ncy, ncz)

    X0 = x_ref[...].reshape(sh)
    Y0 = y_ref[...].reshape(sh)
    Z0 = z_ref[...].reshape(sh)
    c0 = (X0[:, 0:1, 0:1], Y0[:, 0:1, 0:1], Z0[:, 0:1, 0:1])
    lax.fori_loop(0, _S, body, c0)


def _fps_new_xyz(xyz):
    X = xyz[:, :, 0].reshape(_B * _FR, 128)
    Y = xyz[:, :, 1].reshape(_B * _FR, 128)
    Z = xyz[:, :, 2].reshape(_B * _FR, 128)
    out_shape = [jax.ShapeDtypeStruct((_S, _B), jnp.float32)] * 3
    cx, cy, cz = pl.pallas_call(
        _fps_kernel,
        out_shape=out_shape,
        scratch_shapes=[
            pltpu.VMEM((_B * _FR, 128), jnp.float32),
            pltpu.VMEM((_B * _FR, 128), jnp.int32),
        ],
    )(X, Y, Z)
    return jnp.stack([cx.T, cy.T, cz.T], axis=-1)


def _proj_kernel(xyz_ref, feats_ref, nxyz_ref, wxyz_ref, wfeat_ref, p_ref, q_ref):
    xyz = xyz_ref[0]
    feats = feats_ref[0]
    nxyz = nxyz_ref[0]
    wxyz = wxyz_ref[...]
    wfeat = wfeat_ref[...]
    p1 = lax.dot_general(xyz, wxyz, (((1,), (1,)), ((), ())),
                         preferred_element_type=jnp.float32)
    p2 = lax.dot_general(feats, wfeat, (((0,), (1,)), ((), ())),
                         preferred_element_type=jnp.float32)
    p_ref[0] = jnp.concatenate(
        [p1 + p2, jnp.zeros((_N, 96), jnp.float32)], axis=1)
    q_ref[0] = lax.dot_general(nxyz, wxyz, (((1,), (1,)), ((), ())),
                               preferred_element_type=jnp.float32)


def _project(xyz, features, new_xyz, W0):
    wxyz = W0[:, 0:3]
    wfeat = W0[:, 3:]
    p, q = pl.pallas_call(
        _proj_kernel,
        grid=(_B,),
        in_specs=[
            pl.BlockSpec((1, _N, 3), lambda b: (b, 0, 0)),
            pl.BlockSpec((1, _CIN, _N), lambda b: (b, 0, 0)),
            pl.BlockSpec((1, _S, 3), lambda b: (b, 0, 0)),
            pl.BlockSpec((32, 3), lambda b: (0, 0)),
            pl.BlockSpec((32, _CIN), lambda b: (0, 0)),
        ],
        out_specs=[
            pl.BlockSpec((1, _N, 128), lambda b: (b, 0, 0)),
            pl.BlockSpec((1, _S, 32), lambda b: (b, 0, 0)),
        ],
        out_shape=[
            jax.ShapeDtypeStruct((_B, _N, 128), jnp.float32),
            jax.ShapeDtypeStruct((_B, _S, 32), jnp.float32),
        ],
    )(xyz, features, new_xyz, wxyz, wfeat)
    return p, q


_KTILE = 128


def _knn_kernel(nxyz_ref, xyz_ref, idx_ref, d_ref, io_ref):
    c = nxyz_ref[0]
    x = xyz_ref[0]
    s2 = jnp.sum(c * c, axis=1, keepdims=True)
    d2 = jnp.sum(x * x, axis=1, keepdims=True)
    cross = lax.dot_general(c, x, (((1,), (1,)), ((), ())))
    d_ref[...] = jnp.maximum(s2 + d2[:, 0][None, :] - 2.0 * cross, 0.0)
    io_ref[...] = lax.broadcasted_iota(jnp.int32, (_KTILE, _N), 1)

    kiota = lax.broadcasted_iota(jnp.int32, (_KTILE, _K), 1)

    def body(j, acc):
        d = d_ref[...]
        iota = io_ref[...]
        m = jnp.min(d, axis=1, keepdims=True)
        idx = jnp.min(jnp.where(d == m, iota, _N), axis=1, keepdims=True)
        d_ref[...] = jnp.where(iota == idx, jnp.float32(3e38), d)
        return jnp.where(kiota == j, idx, acc)

    idx_ref[0] = lax.fori_loop(
        0, _K, body, jnp.zeros((_KTILE, _K), jnp.int32))


def _knn(new_xyz, xyz):
    return pl.pallas_call(
        _knn_kernel,
        grid=(_B, _S // _KTILE),
        in_specs=[
            pl.BlockSpec((1, _KTILE, 3), lambda b, s: (b, s, 0)),
            pl.BlockSpec((1, _N, 3), lambda b, s: (b, 0, 0)),
        ],
        out_specs=pl.BlockSpec((1, _KTILE, _K), lambda b, s: (b, s, 0)),
        out_shape=jax.ShapeDtypeStruct((_B, _S, _K), jnp.int32),
        scratch_shapes=[
            pltpu.VMEM((_KTILE, _N), jnp.float32),
            pltpu.VMEM((_KTILE, _N), jnp.int32),
        ],
    )(new_xyz, xyz)


_NW = 32
_ROWS_PER_W = _M // _NW
_CHUNK = 256
_NCHUNK = _ROWS_PER_W // _CHUNK


def _sc_gather(p_flat, flat_idx):
    mesh = plsc.VectorSubcoreMesh(core_axis_name="c", subcore_axis_name="s")

    @functools.partial(
        pl.kernel,
        mesh=mesh,
        out_type=jax.ShapeDtypeStruct((_M, 128), jnp.float32),
        scratch_types=[
            pltpu.VMEM((_ROWS_PER_W,), jnp.int32),
            pltpu.VMEM((_CHUNK, 128), jnp.float32),
            pltpu.VMEM((_CHUNK, 128), jnp.float32),
            pltpu.SemaphoreType.DMA,
            pltpu.SemaphoreType.DMA,
        ],
    )
    def k(table_hbm, idx_hbm, out_hbm, idx_all, rows0, rows1, sem0, sem1):
        wid = lax.axis_index("s") * 2 + lax.axis_index("c")
        base = wid * _ROWS_PER_W
        pltpu.sync_copy(idx_hbm.at[pl.ds(base, _ROWS_PER_W)], idx_all)
        rows = (rows0, rows1)
        sems = (sem0, sem1)

        pltpu.async_copy(table_hbm.at[idx_all.at[pl.ds(0, _CHUNK)]],
                         rows0, sem0)
        pltpu.async_copy(table_hbm.at[idx_all.at[pl.ds(_CHUNK, _CHUNK)]],
                         rows1, sem1)

        def body(g, carry):
            for b in range(2):
                j = 2 * g + b
                off = j * _CHUNK
                pltpu.make_async_copy(
                    table_hbm.at[idx_all.at[pl.ds(off, _CHUNK)]],
                    rows[b], sems[b]).wait()
                pltpu.sync_copy(rows[b], out_hbm.at[pl.ds(base + off, _CHUNK)])
                nj = jnp.minimum(j + 2, _NCHUNK - 1)
                pltpu.async_copy(
                    table_hbm.at[idx_all.at[pl.ds(nj * _CHUNK, _CHUNK)]],
                    rows[b], sems[b])
            return carry

        lax.fori_loop(0, _NCHUNK // 2, body, 0)
        tail = pl.ds((_NCHUNK - 1) * _CHUNK, _CHUNK)
        pltpu.make_async_copy(table_hbm.at[idx_all.at[tail]],
                              rows0, sem0).wait()
        pltpu.make_async_copy(table_hbm.at[idx_all.at[tail]],
                              rows1, sem1).wait()

    return k(p_flat, flat_idx)


_RCHUNK = 4096
_NSTEP = _M // _RCHUNK
_SCH = _RCHUNK // _K


def _passA_kernel(g_ref, q_ref, y_ref, st_ref, acc_ref):
    step = pl.program_id(0)

    @pl.when(step == 0)
    def _():
        acc_ref[...] = jnp.zeros_like(acc_ref)

    g = g_ref[:, 0:32]
    q = q_ref[0]
    qrep = jnp.broadcast_to(q[:, None, :], (_SCH, _K, 32)).reshape(_RCHUNK, 32)
    y = g - qrep
    y_ref[...] = y
    acc_ref[0:1, 0:32] += jnp.sum(y, axis=0, keepdims=True)
    acc_ref[1:2, 0:32] += jnp.sum(y * y, axis=0, keepdims=True)

    @pl.when(step == _NSTEP - 1)
    def _():
        st_ref[...] = acc_ref[...]


def _passA(g, q):
    return pl.pallas_call(
        _passA_kernel,
        grid=(_NSTEP,),
        in_specs=[
            pl.BlockSpec((_RCHUNK, 128), lambda i: (i, 0)),
            pl.BlockSpec((1, _SCH, 32), lambda i: (i, 0, 0)),
        ],
        out_specs=[
            pl.BlockSpec((_RCHUNK, 32), lambda i: (i, 0)),
            pl.BlockSpec((8, 128), lambda i: (0, 0)),
        ],
        out_shape=[
            jax.ShapeDtypeStruct((_M, 32), jnp.float32),
            jax.ShapeDtypeStruct((8, 128), jnp.float32),
        ],
        scratch_shapes=[pltpu.VMEM((8, 128), jnp.float32)],
    )(g, q.reshape(_NSTEP, _SCH, 32))


def _norm_relu(y, st_ref, gam_ref, bet_ref, cdim):
    s = st_ref[0:1, 0:cdim]
    ss = st_ref[1:2, 0:cdim]
    mean = s * (1.0 / _M)
    var = ss * (1.0 / _M) - mean * mean
    inv = lax.rsqrt(var + _EPS)
    gam = gam_ref[...]
    bet = bet_ref[...]
    return jnp.maximum((y - mean) * inv * gam + bet, 0.0)


def _passBC_kernel(y_ref, st_ref, gam_ref, bet_ref, w_ref, o_ref, so_ref,
                   acc_ref, *, cin, cout):
    step = pl.program_id(0)

    @pl.when(step == 0)
    def _():
        acc_ref[...] = jnp.zeros_like(acc_ref)

    x = _norm_relu(y_ref[...], st_ref, gam_ref, bet_ref, cin)
    ynew = lax.dot_general(x, w_ref[...], (((1,), (1,)), ((), ())),
                           preferred_element_type=jnp.float32)
    o_ref[...] = ynew
    acc_ref[0:1, 0:cout] += jnp.sum(ynew, axis=0, keepdims=True)
    acc_ref[1:2, 0:cout] += jnp.sum(ynew * ynew, axis=0, keepdims=True)

    @pl.when(step == _NSTEP - 1)
    def _():
        so_ref[...] = acc_ref[...]


def _passBC(y, st, gam, bet, w, cin, cout):
    return pl.pallas_call(
        functools.partial(_passBC_kernel, cin=cin, cout=cout),
        grid=(_NSTEP,),
        in_specs=[
            pl.BlockSpec((_RCHUNK, cin), lambda i: (i, 0)),
            pl.BlockSpec((8, 128), lambda i: (0, 0)),
            pl.BlockSpec((1, cin), lambda i: (0, 0)),
            pl.BlockSpec((1, cin), lambda i: (0, 0)),
            pl.BlockSpec((cout, cin), lambda i: (0, 0)),
        ],
        out_specs=[
            pl.BlockSpec((_RCHUNK, cout), lambda i: (i, 0)),
            pl.BlockSpec((8, 128), lambda i: (0, 0)),
        ],
        out_shape=[
            jax.ShapeDtypeStruct((_M, cout), jnp.float32),
            jax.ShapeDtypeStruct((8, 128), jnp.float32),
        ],
        scratch_shapes=[pltpu.VMEM((8, 128), jnp.float32)],
    )(y, st, gam.reshape(1, cin), bet.reshape(1, cin), w)


def _passD_kernel(y_ref, st_ref, gam_ref, bet_ref, o_ref):
    x = _norm_relu(y_ref[...], st_ref, gam_ref, bet_ref, 64)
    x3 = x.reshape(_SCH, _K, 64)
    o_ref[...] = jnp.max(x3, axis=1)


def _passD(y, st, gam, bet):
    return pl.pallas_call(
        _passD_kernel,
        grid=(_NSTEP,),
        in_specs=[
            pl.BlockSpec((_RCHUNK, 64), lambda i: (i, 0)),
            pl.BlockSpec((8, 128), lambda i: (0, 0)),
            pl.BlockSpec((1, 64), lambda i: (0, 0)),
            pl.BlockSpec((1, 64), lambda i: (0, 0)),
        ],
        out_specs=pl.BlockSpec((_SCH, 64), lambda i: (i, 0)),
        out_shape=jax.ShapeDtypeStruct((_B * _S, 64), jnp.float32),
    )(y, st, gam.reshape(1, 64), bet.reshape(1, 64))


def kernel(xyz, features, W0, g0, b0, W1, g1, b1, W2, g2, b2):
    new_xyz = _fps_new_xyz(xyz)

    group_idx = _knn(new_xyz, xyz)

    p, q = _project(xyz, features, new_xyz, W0)
    flat_idx = (group_idx
                + jnp.arange(_B, dtype=jnp.int32)[:, None, None] * _N
                ).reshape(_M)
    g = _sc_gather(p.reshape(_B * _N, 128), flat_idx)

    y1, st1 = _passA(g, q)
    y2, st2 = _passBC(y1, st1, g0, b0, W1, 32, 32)
    y3, st3 = _passBC(y2, st2, g1, b1, W2, 32, 64)
    nf = _passD(y3, st3, g2, b2)

    new_features = nf.reshape(_B, _S, 64).transpose(0, 2, 1)
    return (new_xyz, new_features)
